# Initial kernel scaffold; baseline (speedup 1.0000x reference)
import jax
import jax.numpy as jnp
from jax.experimental import pallas as pl


def kernel(idx, targets, table):
    raise NotImplementedError("write your pallas kernel here")


# speedup vs baseline: 1.8783x; 1.8783x over previous
import functools

import jax
import jax.numpy as jnp
from jax import lax
from jax.experimental import pallas as pl
from jax.experimental.pallas import tpu as pltpu
from jax.experimental.pallas import tpu_sc as plsc

VOCAB = 8192
NTOK = 8192
NC, NS = 2, 16
NW = NC * NS
B_PER_W = NTOK // NW
CHUNK = 4
NCHUNK = B_PER_W // CHUNK
NBUF = 2

_mesh = plsc.VectorSubcoreMesh(
    core_axis_name="c", subcore_axis_name="s", num_cores=NC, num_subcores=NS
)


@functools.partial(
    pl.kernel,
    out_type=jax.ShapeDtypeStruct((NTOK, VOCAB), jnp.float32),
    mesh=_mesh,
    scratch_types=[
        pltpu.VMEM((NCHUNK, CHUNK), jnp.int32),
        pltpu.VMEM((NBUF, CHUNK, VOCAB), jnp.float32),
        pltpu.SemaphoreType.DMA((NBUF,)),
        pltpu.SemaphoreType.DMA((NBUF,)),
    ],
)
def _sc_gather(idx_hbm, table_hbm, out_hbm, idx_v, bufs, gsem, osem):
    wid = lax.axis_index("s") * NC + lax.axis_index("c")
    base = wid * B_PER_W
    pltpu.sync_copy(idx_hbm.at[wid], idx_v)

    def fire(c, b):
        pltpu.make_async_copy(
            table_hbm.at[idx_v.at[c]], bufs.at[b], gsem.at[b]
        ).start()

    def wait_gather(c, b):
        pltpu.make_async_copy(
            table_hbm.at[idx_v.at[c]], bufs.at[b], gsem.at[b]
        ).wait()

    def out_copy(c, b):
        return pltpu.make_async_copy(
            bufs.at[b], out_hbm.at[pl.ds(base + c * CHUNK, CHUNK)], osem.at[b]
        )

    for b in range(NBUF):
        fire(jnp.int32(b), b)

    def body(k, _):
        for b in range(NBUF):
            c = k * NBUF + b
            wait_gather(c, b)
            out_copy(c, b).start()
            out_copy(c, b).wait()

            @pl.when(c + NBUF < NCHUNK)
            def _():
                fire(c + NBUF, b)

        return 0

    lax.fori_loop(0, NCHUNK // NBUF, body, 0)


_XR = 256
_XNG = NTOK // _XR


def _xent_body(t_ref, x_ref, out_ref):
    i = pl.program_id(0)
    x = x_ref[...]
    m = jnp.max(x, axis=1, keepdims=True)
    s = jnp.sum(jnp.exp(x - m), axis=1)
    logz = m[:, 0] + jnp.log(s)
    t = t_ref[0, 0, :]
    cols = lax.broadcasted_iota(jnp.int32, (_XR, VOCAB), 1)
    tgt = jnp.sum(jnp.where(cols == t[:, None], x, 0.0), axis=1)
    part = jnp.sum(logz - tgt) * (1.0 / NTOK)

    @pl.when(i == 0)
    def _():
        out_ref[...] = jnp.zeros((1, 1), jnp.float32)

    out_ref[...] += part.reshape(1, 1)


_xent = pl.pallas_call(
    _xent_body,
    grid=(_XNG,),
    in_specs=[
        pl.BlockSpec((1, 1, _XR), lambda i: (i, 0, 0)),
        pl.BlockSpec((_XR, VOCAB), lambda i: (i, 0)),
    ],
    out_specs=pl.BlockSpec((1, 1), lambda i: (0, 0)),
    out_shape=jax.ShapeDtypeStruct((1, 1), jnp.float32),
)


def kernel(idx, targets, table):
    idx_flat = idx.reshape(-1).astype(jnp.int32)
    idx3 = idx_flat.reshape(NW, NCHUNK, CHUNK)
    logits2 = _sc_gather(idx3, table)
    t3 = targets.reshape(_XNG, 1, _XR).astype(jnp.int32)
    loss = _xent(t3, logits2)[0, 0]
    return (logits2, loss)


# speedup vs baseline: 1.9259x; 1.0253x over previous
import functools

import jax
import jax.numpy as jnp
from jax import lax
from jax.experimental import pallas as pl
from jax.experimental.pallas import tpu as pltpu
from jax.experimental.pallas import tpu_sc as plsc

VOCAB = 8192
NTOK = 8192
NC, NS = 2, 16
NW = NC * NS
B_PER_W = NTOK // NW
CHUNK = 4
NCHUNK = B_PER_W // CHUNK
NBUF = 3

_mesh = plsc.VectorSubcoreMesh(
    core_axis_name="c", subcore_axis_name="s", num_cores=NC, num_subcores=NS
)


@functools.partial(
    pl.kernel,
    out_type=jax.ShapeDtypeStruct((NTOK, VOCAB), jnp.float32),
    mesh=_mesh,
    scratch_types=[
        pltpu.VMEM((NCHUNK, CHUNK), jnp.int32),
        pltpu.VMEM((NBUF, CHUNK, VOCAB), jnp.float32),
        pltpu.SemaphoreType.DMA((NBUF,)),
        pltpu.SemaphoreType.DMA((NBUF,)),
    ],
)
def _sc_gather(idx_hbm, table_hbm, out_hbm, idx_v, bufs, gsem, osem):
    wid = lax.axis_index("s") * NC + lax.axis_index("c")
    base = wid * B_PER_W
    pltpu.sync_copy(idx_hbm.at[wid], idx_v)

    def fire(c, b):
        pltpu.make_async_copy(
            table_hbm.at[idx_v.at[c]], bufs.at[b], gsem.at[b]
        ).start()

    def wait_gather(c, b):
        pltpu.make_async_copy(
            table_hbm.at[idx_v.at[c]], bufs.at[b], gsem.at[b]
        ).wait()

    def out_copy(c, b):
        return pltpu.make_async_copy(
            bufs.at[b], out_hbm.at[pl.ds(base + c * CHUNK, CHUNK)], osem.at[b]
        )

    for b in range(NBUF):
        fire(jnp.int32(b), b)

    def body(k, _):
        for b in range(NBUF):
            c = k * NBUF + b
            bf = (b - 1) % NBUF
            f = c + NBUF - 1

            @pl.when(jnp.logical_and(c >= 1, f < NCHUNK))
            def _():
                out_copy(f - NBUF, bf).wait()
                fire(f, bf)

            wait_gather(c, b)
            out_copy(c, b).start()

        return 0

    lax.fori_loop(0, (NCHUNK - 1) // NBUF, body, 0)
    last = NCHUNK - 1
    wait_gather(jnp.int32(last), last % NBUF)
    out_copy(jnp.int32(last), last % NBUF).start()
    for c in range(NCHUNK - NBUF, NCHUNK):
        out_copy(jnp.int32(c), c % NBUF).wait()


_XR = 512
_XNG = NTOK // _XR


def _xent_body(t_ref, x_ref, out_ref):
    i = pl.program_id(0)
    x = x_ref[...]
    m = jnp.max(x, axis=1, keepdims=True)
    s = jnp.sum(jnp.exp(x - m), axis=1)
    logz = m[:, 0] + jnp.log(s)
    t = t_ref[0, 0, :]
    cols = lax.broadcasted_iota(jnp.int32, (_XR, VOCAB), 1)
    tgt = jnp.sum(jnp.where(cols == t[:, None], x, 0.0), axis=1)
    part = jnp.sum(logz - tgt) * (1.0 / NTOK)

    @pl.when(i == 0)
    def _():
        out_ref[...] = jnp.zeros((1, 1), jnp.float32)

    out_ref[...] += part.reshape(1, 1)


_xent = pl.pallas_call(
    _xent_body,
    grid=(_XNG,),
    in_specs=[
        pl.BlockSpec((1, 1, _XR), lambda i: (i, 0, 0)),
        pl.BlockSpec((_XR, VOCAB), lambda i: (i, 0)),
    ],
    out_specs=pl.BlockSpec((1, 1), lambda i: (0, 0)),
    out_shape=jax.ShapeDtypeStruct((1, 1), jnp.float32),
)


def kernel(idx, targets, table):
    idx_flat = idx.reshape(-1).astype(jnp.int32)
    idx3 = idx_flat.reshape(NW, NCHUNK, CHUNK)
    logits2 = _sc_gather(idx3, table)
    t3 = targets.reshape(_XNG, 1, _XR).astype(jnp.int32)
    loss = _xent(t3, logits2)[0, 0]
    return (logits2, loss)
